# baseline (speedup 1.0000x reference)
import functools

import jax
import jax.numpy as jnp
from jax import lax
from jax.experimental import pallas as pl
from jax.experimental.pallas import tpu as pltpu
from jax.experimental.pallas import tpu_sc as plsc

B, T, K, D = 1024, 50, 8, 128
BT = B * T
L = 16
C = 16
NROW = 2 + 3 * K


def _proj_body(a_ref, w_ref, b_ref, o_ref, *, zero_first: bool, block_rows: int):
    a = a_ref[...]
    if zero_first:
        row = lax.broadcasted_iota(jnp.int32, a.shape, 0) + pl.program_id(0) * block_rows
        a = jnp.where(row == 0, 0.0, a)
    o_ref[...] = jnp.dot(a, w_ref[...], preferred_element_type=jnp.float32) + b_ref[...]


def _project(table, wblk, bias, zero_first):
    n = table.shape[0]
    r = 2000 if n % 2000 == 0 else n
    grid = n // r
    return pl.pallas_call(
        functools.partial(_proj_body, zero_first=zero_first, block_rows=r),
        grid=(grid,),
        in_specs=[
            pl.BlockSpec((r, D), lambda i: (i, 0)),
            pl.BlockSpec((D, D), lambda i: (0, 0)),
            pl.BlockSpec((1, D), lambda i: (0, 0)),
        ],
        out_specs=pl.BlockSpec((r, D), lambda i: (i, 0)),
        out_shape=jax.ShapeDtypeStruct((n, D), jnp.float32),
    )(table, wblk, bias)


def _make_sc_encode(nc, ns):
    nw = nc * ns
    cpw = BT // nw
    nchunk = cpw // C

    mesh = plsc.VectorSubcoreMesh(core_axis_name="c", subcore_axis_name="s")

    @functools.partial(
        pl.kernel,
        mesh=mesh,
        out_type=jax.ShapeDtypeStruct((BT, D), jnp.float32),
        scratch_types=[
            pltpu.VMEM((NROW, C), jnp.int32),
            pltpu.VMEM((NROW, C), jnp.int32),
            pltpu.VMEM((NROW, C, D), jnp.float32),
            pltpu.VMEM((NROW, C, D), jnp.float32),
            pltpu.VMEM((3, C + L), jnp.int32),
            pltpu.VMEM((9, L), jnp.float32),
            pltpu.VMEM((C, D), jnp.float32),
            pltpu.SemaphoreType.DMA,
            pltpu.SemaphoreType.DMA,
            pltpu.SemaphoreType.DMA,
        ],
    )
    def sc_encode(pet, pac, pa, pth, pco, ipack, invtab, out_hbm,
                  ipk0, ipk1, rows0, rows1, cntb, invt, ob, sem0, sem1, osem):
        wid = lax.axis_index("s") * nc + lax.axis_index("c")
        pltpu.sync_copy(invtab, invt)
        tables = [pet, pac] + [pa] * K + [pth] * K + [pco] * K
        ipks = (ipk0, ipk1)
        rowss = (rows0, rows1)
        sems = (sem0, sem1)

        def issue(cid, slot):
            ipk, rows, sem = ipks[slot], rowss[slot], sems[slot]
            pltpu.sync_copy(ipack.at[cid], ipk)
            for j, tab in enumerate(tables):
                pltpu.async_copy(tab.at[ipk.at[j]], rows.at[j], sem)

        def drain(slot):
            ipk, rows, sem = ipks[slot], rowss[slot], sems[slot]
            for j, tab in enumerate(tables):
                pltpu.make_async_copy(tab.at[ipk.at[j]], rows.at[j], sem).wait()

        def compute(cid, slot):
            ipk, rows = ipks[slot], rowss[slot]
            for f in range(3):
                for tg in range(C // L):
                    sl = pl.ds(tg * L, L)
                    cnt = jnp.zeros((L,), jnp.int32)
                    for k in range(K):
                        cnt = cnt + jnp.where(ipk[2 + f * K + k, sl] != 0, 1, 0)
                    cntb[f, sl] = cnt
            drain(slot)

            def tok(t, c):
                ibs = []
                for f in range(3):
                    cv = cntb[f, pl.ds(t, L)]
                    ibs.append(invt[cv[0], :])
                for d in range(D // L):
                    sl = pl.ds(d * L, L)
                    v = rows[0, t, sl] + rows[1, t, sl]
                    for f in range(3):
                        acc = rows[2 + f * K, t, sl]
                        for k in range(1, K):
                            acc = acc + rows[2 + f * K + k, t, sl]
                        v = v + acc * ibs[f]
                    ob[t, sl] = v
                return c

            lax.fori_loop(0, C, tok, 0)

        def store(cid):
            pltpu.async_copy(ob, out_hbm.at[pl.ds(cid * C, C)], osem)

        def wait_store(cid):
            pltpu.make_async_copy(ob, out_hbm.at[pl.ds(cid * C, C)], osem).wait()

        base = wid * nchunk
        issue(base, 0)

        def outer(i, carry):
            for b in range(2):
                g = i * 2 + b
                cid = base + g

                @pl.when(g + 1 < nchunk)
                def _():
                    issue(cid + 1, 1 - b)

                @pl.when(g > 0)
                def _():
                    wait_store(cid)
                compute(cid, b)
                store(cid)
            return carry

        lax.fori_loop(0, nchunk // 2, outer, 0)
        wait_store(base)

    return sc_encode


def kernel(event_type, action, actors, themes, constraints,
           event_type_emb, action_emb, actor_emb, theme_emb, constraint_emb,
           W, b):
    wr = W.reshape(5, D, D)
    zero_bias = jnp.zeros((1, D), jnp.float32)
    pet = _project(event_type_emb, wr[0], b.reshape(1, D), False)
    pac = _project(action_emb, wr[1], zero_bias, False)
    pa = _project(actor_emb, wr[2], zero_bias, True)
    pth = _project(theme_emb, wr[3], zero_bias, True)
    pco = _project(constraint_emb, wr[4], zero_bias, True)

    info = plsc.get_sparse_core_info()
    nw = info.num_cores * info.num_subcores
    cpw = BT // nw
    nchunk = cpw // C

    def chunk2(x):
        return x.reshape(nw, nchunk, 1, C)

    def chunk3(x):
        return x.reshape(nw, nchunk, C, K).transpose(0, 1, 3, 2)

    ipack = jnp.concatenate(
        [chunk2(event_type), chunk2(action),
         chunk3(actors), chunk3(themes), chunk3(constraints)], axis=2,
    ).reshape(nw * nchunk, NROW, C)

    invtab = jnp.broadcast_to(
        (1.0 / jnp.maximum(jnp.arange(9, dtype=jnp.float32), 1.0))[:, None], (9, L))

    sc_encode = _make_sc_encode(info.num_cores, info.num_subcores)
    out = sc_encode(pet, pac, pa, pth, pco, ipack, invtab)
    return out.reshape(B, T, D)


# speedup vs baseline: 1.3814x; 1.3814x over previous
import functools

import jax
import jax.numpy as jnp
from jax import lax
from jax.experimental import pallas as pl
from jax.experimental.pallas import tpu as pltpu
from jax.experimental.pallas import tpu_sc as plsc

B, T, K, D = 1024, 50, 8, 128
BT = B * T
L = 16
C = 16
CK = C * K
NIDX = 2 * C + 3 * CK


def _proj_body(a_ref, w_ref, b_ref, o_ref, *, zero_first: bool, block_rows: int):
    a = a_ref[...]
    if zero_first:
        row = lax.broadcasted_iota(jnp.int32, a.shape, 0) + pl.program_id(0) * block_rows
        a = jnp.where(row == 0, 0.0, a)
    o_ref[...] = jnp.dot(a, w_ref[...], preferred_element_type=jnp.float32) + b_ref[...]


def _project(table, wblk, bias, zero_first):
    n = table.shape[0]
    r = 2000 if n % 2000 == 0 else n
    grid = n // r
    return pl.pallas_call(
        functools.partial(_proj_body, zero_first=zero_first, block_rows=r),
        grid=(grid,),
        in_specs=[
            pl.BlockSpec((r, D), lambda i: (i, 0)),
            pl.BlockSpec((D, D), lambda i: (0, 0)),
            pl.BlockSpec((1, D), lambda i: (0, 0)),
        ],
        out_specs=pl.BlockSpec((r, D), lambda i: (i, 0)),
        out_shape=jax.ShapeDtypeStruct((n, D), jnp.float32),
    )(table, wblk, bias)


def _inv_body(a_ref, t_ref, c_ref, oa_ref, ot_ref, oc_ref):
    for x_ref, o_ref in ((a_ref, oa_ref), (t_ref, ot_ref), (c_ref, oc_ref)):
        cnt = jnp.sum((x_ref[...] != 0).astype(jnp.float32), axis=1)
        inv = 1.0 / jnp.maximum(cnt, 1.0)
        o_ref[...] = jnp.broadcast_to(inv[:, None], (inv.shape[0], L))


def _inv_splats(actors, themes, constraints):
    rb = 6400
    grid = BT // rb
    spec_in = pl.BlockSpec((rb, K), lambda i: (i, 0))
    spec_out = pl.BlockSpec((rb, L), lambda i: (i, 0))
    shape = jax.ShapeDtypeStruct((BT, L), jnp.float32)
    return pl.pallas_call(
        _inv_body,
        grid=(grid,),
        in_specs=[spec_in, spec_in, spec_in],
        out_specs=[spec_out, spec_out, spec_out],
        out_shape=[shape, shape, shape],
    )(actors.reshape(BT, K), themes.reshape(BT, K), constraints.reshape(BT, K))


def _make_sc_encode(nc, ns):
    nw = nc * ns
    cpw = BT // nw
    nchunk = cpw // C

    mesh = plsc.VectorSubcoreMesh(core_axis_name="c", subcore_axis_name="s")

    @functools.partial(
        pl.kernel,
        mesh=mesh,
        out_type=jax.ShapeDtypeStruct((BT, D), jnp.float32),
        scratch_types=[
            pltpu.VMEM((NIDX,), jnp.int32),
            pltpu.VMEM((NIDX,), jnp.int32),
            pltpu.VMEM((NIDX, D), jnp.float32),
            pltpu.VMEM((NIDX, D), jnp.float32),
            pltpu.VMEM((3, C, L), jnp.float32),
            pltpu.VMEM((3, C, L), jnp.float32),
            pltpu.VMEM((C, D), jnp.float32),
            pltpu.SemaphoreType.DMA,
            pltpu.SemaphoreType.DMA,
            pltpu.SemaphoreType.DMA,
        ],
    )
    def sc_encode(pet, pac, pa, pth, pco, ipack, inva, invth, invco, out_hbm,
                  ipk0, ipk1, rows0, rows1, inv0, inv1, ob, sem0, sem1, osem):
        wid = lax.axis_index("s") * nc + lax.axis_index("c")
        fields = [(0, C, pet), (C, C, pac), (2 * C, CK, pa),
                  (2 * C + CK, CK, pth), (2 * C + 2 * CK, CK, pco)]
        invsrc = (inva, invth, invco)
        ipks = (ipk0, ipk1)
        rowss = (rows0, rows1)
        invb = (inv0, inv1)
        sems = (sem0, sem1)

        def issue(cid, slot):
            ipk, rows, inv, sem = ipks[slot], rowss[slot], invb[slot], sems[slot]
            pltpu.sync_copy(ipack.at[cid], ipk)
            for off, n, tab in fields:
                pltpu.async_copy(tab.at[ipk.at[pl.ds(off, n)]],
                                 rows.at[pl.ds(off, n)], sem)
            for f in range(3):
                pltpu.async_copy(invsrc[f].at[cid], inv.at[f], sem)

        def drain(cid, slot):
            ipk, rows, inv, sem = ipks[slot], rowss[slot], invb[slot], sems[slot]
            for off, n, tab in fields:
                pltpu.make_async_copy(tab.at[ipk.at[pl.ds(off, n)]],
                                      rows.at[pl.ds(off, n)], sem).wait()
            for f in range(3):
                pltpu.make_async_copy(invsrc[f].at[cid], inv.at[f], sem).wait()

        def compute(cid, slot):
            rows, inv = rowss[slot], invb[slot]
            drain(cid, slot)

            def tok(t, c):
                ivs = [inv[f, t, :] for f in range(3)]
                for d in range(D // L):
                    sl = pl.ds(d * L, L)
                    v = rows[t, sl] + rows[C + t, sl]
                    for f in range(3):
                        base = 2 * C + f * CK + t * K
                        acc = rows[base, sl]
                        for k in range(1, K):
                            acc = acc + rows[base + k, sl]
                        v = v + acc * ivs[f]
                    ob[t, sl] = v
                return c

            lax.fori_loop(0, C, tok, 0)

        def store(cid):
            pltpu.async_copy(ob, out_hbm.at[pl.ds(cid * C, C)], osem)

        def wait_store(cid):
            pltpu.make_async_copy(ob, out_hbm.at[pl.ds(cid * C, C)], osem).wait()

        base = wid * nchunk
        issue(base, 0)

        def outer(i, carry):
            for b in range(2):
                g = i * 2 + b
                cid = base + g

                @pl.when(g + 1 < nchunk)
                def _():
                    issue(cid + 1, 1 - b)

                @pl.when(g > 0)
                def _():
                    wait_store(cid)
                compute(cid, b)
                store(cid)
            return carry

        lax.fori_loop(0, nchunk // 2, outer, 0)
        wait_store(base)

    return sc_encode


def kernel(event_type, action, actors, themes, constraints,
           event_type_emb, action_emb, actor_emb, theme_emb, constraint_emb,
           W, b):
    wr = W.reshape(5, D, D)
    zero_bias = jnp.zeros((1, D), jnp.float32)
    pet = _project(event_type_emb, wr[0], b.reshape(1, D), False)
    pac = _project(action_emb, wr[1], zero_bias, False)
    pa = _project(actor_emb, wr[2], zero_bias, True)
    pth = _project(theme_emb, wr[3], zero_bias, True)
    pco = _project(constraint_emb, wr[4], zero_bias, True)

    info = plsc.get_sparse_core_info()
    nw = info.num_cores * info.num_subcores
    nchunks = BT // C

    ipack = jnp.concatenate(
        [event_type.reshape(nchunks, C), action.reshape(nchunks, C),
         actors.reshape(nchunks, CK), themes.reshape(nchunks, CK),
         constraints.reshape(nchunks, CK)], axis=1)

    inva, invth, invco = _inv_splats(actors, themes, constraints)

    sc_encode = _make_sc_encode(info.num_cores, info.num_subcores)
    out = sc_encode(pet, pac, pa, pth, pco, ipack,
                    inva.reshape(nchunks, C, L), invth.reshape(nchunks, C, L),
                    invco.reshape(nchunks, C, L))
    return out.reshape(B, T, D)


# speedup vs baseline: 1.4792x; 1.0708x over previous
import functools

import jax
import jax.numpy as jnp
from jax import lax
from jax.experimental import pallas as pl
from jax.experimental.pallas import tpu as pltpu
from jax.experimental.pallas import tpu_sc as plsc

B, T, K, D = 1024, 50, 8, 128
BT = B * T
L = 16
C = 16
CK = C * K
NIDX = 2 * C + 3 * CK


def _proj_body(a_ref, w_ref, b_ref, o_ref, *, zero_first: bool, block_rows: int):
    a = a_ref[...]
    if zero_first:
        row = lax.broadcasted_iota(jnp.int32, a.shape, 0) + pl.program_id(0) * block_rows
        a = jnp.where(row == 0, 0.0, a)
    o_ref[...] = jnp.dot(a, w_ref[...], preferred_element_type=jnp.float32) + b_ref[...]


def _project(table, wblk, bias, zero_first):
    n = table.shape[0]
    r = 2000 if n % 2000 == 0 else n
    grid = n // r
    return pl.pallas_call(
        functools.partial(_proj_body, zero_first=zero_first, block_rows=r),
        grid=(grid,),
        in_specs=[
            pl.BlockSpec((r, D), lambda i: (i, 0)),
            pl.BlockSpec((D, D), lambda i: (0, 0)),
            pl.BlockSpec((1, D), lambda i: (0, 0)),
        ],
        out_specs=pl.BlockSpec((r, D), lambda i: (i, 0)),
        out_shape=jax.ShapeDtypeStruct((n, D), jnp.float32),
    )(table, wblk, bias)


def _cnt_body(a_ref, t_ref, c_ref, o_ref):
    for f, x_ref in enumerate((a_ref, t_ref, c_ref)):
        x = x_ref[...]
        cnt = jnp.sum((x.reshape(x.shape[0], C, K) != 0).astype(jnp.int32), axis=2)
        o_ref[:, f, :] = jnp.concatenate([cnt, cnt], axis=1)


def _cnt_chunks(actors, themes, constraints):
    nchunks = BT // C
    rc = 400
    grid = nchunks // rc
    spec_in = pl.BlockSpec((rc, CK), lambda i: (i, 0))
    return pl.pallas_call(
        _cnt_body,
        grid=(grid,),
        in_specs=[spec_in, spec_in, spec_in],
        out_specs=pl.BlockSpec((rc, 3, 2 * L), lambda i: (i, 0, 0)),
        out_shape=jax.ShapeDtypeStruct((nchunks, 3, 2 * L), jnp.int32),
    )(actors.reshape(nchunks, CK), themes.reshape(nchunks, CK),
      constraints.reshape(nchunks, CK))


def _make_sc_encode(nc, ns):
    nw = nc * ns
    cpw = BT // nw
    nchunk = cpw // C

    mesh = plsc.VectorSubcoreMesh(core_axis_name="c", subcore_axis_name="s")

    @functools.partial(
        pl.kernel,
        mesh=mesh,
        out_type=jax.ShapeDtypeStruct((BT, D), jnp.float32),
        scratch_types=[
            pltpu.VMEM((NIDX,), jnp.int32),
            pltpu.VMEM((NIDX,), jnp.int32),
            pltpu.VMEM((NIDX, D), jnp.float32),
            pltpu.VMEM((NIDX, D), jnp.float32),
            pltpu.VMEM((3, 2 * L), jnp.int32),
            pltpu.VMEM((3, 2 * L), jnp.int32),
            pltpu.VMEM((9, L), jnp.float32),
            pltpu.VMEM((C, D), jnp.float32),
            pltpu.SemaphoreType.DMA,
            pltpu.SemaphoreType.DMA,
            pltpu.SemaphoreType.DMA,
        ],
    )
    def sc_encode(pet, pac, pa, pth, pco, ipack, cnts, invtab, out_hbm,
                  ipk0, ipk1, rows0, rows1, cnt0, cnt1, invt, ob,
                  sem0, sem1, osem):
        wid = lax.axis_index("s") * nc + lax.axis_index("c")
        pltpu.sync_copy(invtab, invt)
        fields = [(0, C, pet), (C, C, pac), (2 * C, CK, pa),
                  (2 * C + CK, CK, pth), (2 * C + 2 * CK, CK, pco)]
        ipks = (ipk0, ipk1)
        rowss = (rows0, rows1)
        cntb = (cnt0, cnt1)
        sems = (sem0, sem1)

        def issue(cid, slot):
            ipk, rows, cnt, sem = ipks[slot], rowss[slot], cntb[slot], sems[slot]
            pltpu.sync_copy(ipack.at[cid], ipk)
            for off, n, tab in fields:
                pltpu.async_copy(tab.at[ipk.at[pl.ds(off, n)]],
                                 rows.at[pl.ds(off, n)], sem)
            pltpu.async_copy(cnts.at[cid], cnt, sem)

        def drain(cid, slot):
            ipk, rows, cnt, sem = ipks[slot], rowss[slot], cntb[slot], sems[slot]
            for off, n, tab in fields:
                pltpu.make_async_copy(tab.at[ipk.at[pl.ds(off, n)]],
                                      rows.at[pl.ds(off, n)], sem).wait()
            pltpu.make_async_copy(cnts.at[cid], cnt, sem).wait()

        def compute(cid, slot):
            rows, cnt = rowss[slot], cntb[slot]
            drain(cid, slot)

            def tok(t, c):
                ivs = []
                for f in range(3):
                    cv = cnt[f, pl.ds(t, L)]
                    ivs.append(invt[cv[0], :])
                for d in range(D // L):
                    sl = pl.ds(d * L, L)
                    v = rows[t, sl] + rows[C + t, sl]
                    for f in range(3):
                        base = 2 * C + f * CK + t * K
                        acc = rows[base, sl]
                        for k in range(1, K):
                            acc = acc + rows[base + k, sl]
                        v = v + acc * ivs[f]
                    ob[t, sl] = v
                return c

            lax.fori_loop(0, C, tok, 0)

        def store(cid):
            pltpu.async_copy(ob, out_hbm.at[pl.ds(cid * C, C)], osem)

        def wait_store(cid):
            pltpu.make_async_copy(ob, out_hbm.at[pl.ds(cid * C, C)], osem).wait()

        base = wid * nchunk
        issue(base, 0)

        def outer(i, carry):
            for b in range(2):
                g = i * 2 + b
                cid = base + g

                @pl.when(g + 1 < nchunk)
                def _():
                    issue(cid + 1, 1 - b)

                @pl.when(g > 0)
                def _():
                    wait_store(cid)
                compute(cid, b)
                store(cid)
            return carry

        lax.fori_loop(0, nchunk // 2, outer, 0)
        wait_store(base)

    return sc_encode


def kernel(event_type, action, actors, themes, constraints,
           event_type_emb, action_emb, actor_emb, theme_emb, constraint_emb,
           W, b):
    wr = W.reshape(5, D, D)
    zero_bias = jnp.zeros((1, D), jnp.float32)
    pet = _project(event_type_emb, wr[0], b.reshape(1, D), False)
    pac = _project(action_emb, wr[1], zero_bias, False)
    pa = _project(actor_emb, wr[2], zero_bias, True)
    pth = _project(theme_emb, wr[3], zero_bias, True)
    pco = _project(constraint_emb, wr[4], zero_bias, True)

    info = plsc.get_sparse_core_info()
    nw = info.num_cores * info.num_subcores
    nchunks = BT // C

    ipack = jnp.concatenate(
        [event_type.reshape(nchunks, C), action.reshape(nchunks, C),
         actors.reshape(nchunks, CK), themes.reshape(nchunks, CK),
         constraints.reshape(nchunks, CK)], axis=1)

    cnts = _cnt_chunks(actors, themes, constraints)

    invtab = jnp.broadcast_to(
        (1.0 / jnp.maximum(jnp.arange(9, dtype=jnp.float32), 1.0))[:, None], (9, L))

    sc_encode = _make_sc_encode(info.num_cores, info.num_subcores)
    out = sc_encode(pet, pac, pa, pth, pco, ipack, cnts, invtab)
    return out.reshape(B, T, D)


# speedup vs baseline: 1.6071x; 1.0864x over previous
import functools

import jax
import jax.numpy as jnp
from jax import lax
from jax.experimental import pallas as pl
from jax.experimental.pallas import tpu as pltpu
from jax.experimental.pallas import tpu_sc as plsc

B, T, K, D = 1024, 50, 8, 128
BT = B * T
L = 16
C = 16
CK = C * K
NIDX = 2 * C + 3 * CK


def _proj_body(a_ref, w_ref, b_ref, o_ref, *, zero_first: bool, block_rows: int):
    a = a_ref[...]
    if zero_first:
        row = lax.broadcasted_iota(jnp.int32, a.shape, 0) + pl.program_id(0) * block_rows
        a = jnp.where(row == 0, 0.0, a)
    o_ref[...] = jnp.dot(a, w_ref[...], preferred_element_type=jnp.float32) + b_ref[...]


def _project(table, wblk, bias, zero_first):
    n = table.shape[0]
    r = 2000 if n % 2000 == 0 else n
    grid = n // r
    return pl.pallas_call(
        functools.partial(_proj_body, zero_first=zero_first, block_rows=r),
        grid=(grid,),
        in_specs=[
            pl.BlockSpec((r, D), lambda i: (i, 0)),
            pl.BlockSpec((D, D), lambda i: (0, 0)),
            pl.BlockSpec((1, D), lambda i: (0, 0)),
        ],
        out_specs=pl.BlockSpec((r, D), lambda i: (i, 0)),
        out_shape=jax.ShapeDtypeStruct((n, D), jnp.float32),
    )(table, wblk, bias)


def _cnt_body(a_ref, t_ref, c_ref, g_ref, o_ref):
    g = g_ref[...]
    for f, x_ref in enumerate((a_ref, t_ref, c_ref)):
        m = (x_ref[...] != 0).astype(jnp.float32)
        cnt = jnp.dot(m, g, preferred_element_type=jnp.float32)
        o_ref[:, f, :] = cnt.astype(jnp.int32)


def _cnt_chunks(actors, themes, constraints):
    nchunks = BT // C
    rc = 400
    grid = nchunks // rc
    i = jnp.arange(CK)[:, None]
    j = jnp.arange(2 * L)[None, :]
    gmat = (i // K == j % C).astype(jnp.float32)
    spec_in = pl.BlockSpec((rc, CK), lambda i: (i, 0))
    return pl.pallas_call(
        _cnt_body,
        grid=(grid,),
        in_specs=[spec_in, spec_in, spec_in,
                  pl.BlockSpec((CK, 2 * L), lambda i: (0, 0))],
        out_specs=pl.BlockSpec((rc, 3, 2 * L), lambda i: (i, 0, 0)),
        out_shape=jax.ShapeDtypeStruct((nchunks, 3, 2 * L), jnp.int32),
    )(actors.reshape(nchunks, CK), themes.reshape(nchunks, CK),
      constraints.reshape(nchunks, CK), gmat)


def _make_sc_encode(nc, ns):
    nw = nc * ns
    cpw = BT // nw
    nchunk = cpw // C

    mesh = plsc.VectorSubcoreMesh(core_axis_name="c", subcore_axis_name="s")

    @functools.partial(
        pl.kernel,
        mesh=mesh,
        out_type=jax.ShapeDtypeStruct((BT, D), jnp.float32),
        scratch_types=[
            pltpu.VMEM((NIDX,), jnp.int32),
            pltpu.VMEM((NIDX,), jnp.int32),
            pltpu.VMEM((NIDX, D), jnp.float32),
            pltpu.VMEM((NIDX, D), jnp.float32),
            pltpu.VMEM((3, 2 * L), jnp.int32),
            pltpu.VMEM((3, 2 * L), jnp.int32),
            pltpu.VMEM((9, L), jnp.float32),
            pltpu.VMEM((C, D), jnp.float32),
            pltpu.SemaphoreType.DMA,
            pltpu.SemaphoreType.DMA,
            pltpu.SemaphoreType.DMA,
        ],
    )
    def sc_encode(pet, pac, pa, pth, pco, ipack, cnts, invtab, out_hbm,
                  ipk0, ipk1, rows0, rows1, cnt0, cnt1, invt, ob,
                  sem0, sem1, osem):
        wid = lax.axis_index("s") * nc + lax.axis_index("c")
        pltpu.sync_copy(invtab, invt)
        fields = [(0, C, pet), (C, C, pac), (2 * C, CK, pa),
                  (2 * C + CK, CK, pth), (2 * C + 2 * CK, CK, pco)]
        ipks = (ipk0, ipk1)
        rowss = (rows0, rows1)
        cntb = (cnt0, cnt1)
        sems = (sem0, sem1)

        def issue(cid, slot):
            ipk, rows, cnt, sem = ipks[slot], rowss[slot], cntb[slot], sems[slot]
            pltpu.sync_copy(ipack.at[cid], ipk)
            for off, n, tab in fields:
                pltpu.async_copy(tab.at[ipk.at[pl.ds(off, n)]],
                                 rows.at[pl.ds(off, n)], sem)
            pltpu.async_copy(cnts.at[cid], cnt, sem)

        def drain(cid, slot):
            ipk, rows, cnt, sem = ipks[slot], rowss[slot], cntb[slot], sems[slot]
            for off, n, tab in fields:
                pltpu.make_async_copy(tab.at[ipk.at[pl.ds(off, n)]],
                                      rows.at[pl.ds(off, n)], sem).wait()
            pltpu.make_async_copy(cnts.at[cid], cnt, sem).wait()

        def compute(cid, slot):
            rows, cnt = rowss[slot], cntb[slot]
            drain(cid, slot)

            def tok(t, c):
                ivs = []
                for f in range(3):
                    cv = cnt[f, pl.ds(t, L)]
                    ivs.append(invt[cv[0], :])
                for d in range(D // L):
                    sl = pl.ds(d * L, L)
                    v = rows[t, sl] + rows[C + t, sl]
                    for f in range(3):
                        base = 2 * C + f * CK + t * K
                        acc = rows[base, sl]
                        for k in range(1, K):
                            acc = acc + rows[base + k, sl]
                        v = v + acc * ivs[f]
                    ob[t, sl] = v
                return c

            lax.fori_loop(0, C, tok, 0)

        def store(cid):
            pltpu.async_copy(ob, out_hbm.at[pl.ds(cid * C, C)], osem)

        def wait_store(cid):
            pltpu.make_async_copy(ob, out_hbm.at[pl.ds(cid * C, C)], osem).wait()

        base = wid * nchunk
        issue(base, 0)

        def outer(i, carry):
            for b in range(2):
                g = i * 2 + b
                cid = base + g

                @pl.when(g + 1 < nchunk)
                def _():
                    issue(cid + 1, 1 - b)

                @pl.when(g > 0)
                def _():
                    wait_store(cid)
                compute(cid, b)
                store(cid)
            return carry

        lax.fori_loop(0, nchunk // 2, outer, 0)
        wait_store(base)

    return sc_encode


def kernel(event_type, action, actors, themes, constraints,
           event_type_emb, action_emb, actor_emb, theme_emb, constraint_emb,
           W, b):
    wr = W.reshape(5, D, D)
    zero_bias = jnp.zeros((1, D), jnp.float32)
    pet = _project(event_type_emb, wr[0], b.reshape(1, D), False)
    pac = _project(action_emb, wr[1], zero_bias, False)
    pa = _project(actor_emb, wr[2], zero_bias, True)
    pth = _project(theme_emb, wr[3], zero_bias, True)
    pco = _project(constraint_emb, wr[4], zero_bias, True)

    info = plsc.get_sparse_core_info()
    nw = info.num_cores * info.num_subcores
    nchunks = BT // C

    ipack = jnp.concatenate(
        [event_type.reshape(nchunks, C), action.reshape(nchunks, C),
         actors.reshape(nchunks, CK), themes.reshape(nchunks, CK),
         constraints.reshape(nchunks, CK)], axis=1)

    cnts = _cnt_chunks(actors, themes, constraints)

    invtab = jnp.broadcast_to(
        (1.0 / jnp.maximum(jnp.arange(9, dtype=jnp.float32), 1.0))[:, None], (9, L))

    sc_encode = _make_sc_encode(info.num_cores, info.num_subcores)
    out = sc_encode(pet, pac, pa, pth, pco, ipack, cnts, invtab)
    return out.reshape(B, T, D)
